# baseline (speedup 1.0000x reference)
import jax
import jax.numpy as jnp
from jax.experimental import pallas as pl
from jax.experimental.pallas import tpu as pltpu

NUM_FEATURES = 128
NUM_PIDS = 100000
NUM_CQ = 5000
OIM_SCALAR = 30.0
BATCH = 128
BLK = 10000
NUM_STEPS = 5
IGNORE_INDEX = 5554


def _oim_kernel(x_ref, lab_ref, luta_ref, lutb_ref, cq_ref,
                rela_ref, relb_ref, rel_cq_ref,
                out_ref, s_ref, lsc_ref):
    i = pl.program_id(0)
    x = x_ref[...]
    labels = lab_ref[...]

    def scores(w, rel):
        lg = jax.lax.dot_general(
            x, w, (((1,), (1,)), ((), ())),
            preferred_element_type=jnp.float32,
            precision=jax.lax.Precision.DEFAULT)
        return lg * (rel * OIM_SCALAR)

    def accum(w, rel, base):
        ls = scores(w, rel)
        se = jnp.sum(jnp.exp(ls), axis=1, keepdims=True)
        col = jax.lax.broadcasted_iota(jnp.int32, ls.shape, 1) + base
        hit = col == labels
        lsum = jnp.sum(jnp.where(hit, ls, 0.0), axis=1, keepdims=True)
        return se, lsum

    @pl.when(i == 0)
    def _init():
        cs = scores(cq_ref[...], rel_cq_ref[...])
        s_ref[...] = jnp.sum(jnp.exp(cs), axis=1, keepdims=True)
        lsc_ref[...] = jnp.zeros_like(lsc_ref)

    se_a, lsum_a = accum(luta_ref[...], rela_ref[0], i * BLK)
    se_b, lsum_b = accum(lutb_ref[...], relb_ref[0],
                         (i + NUM_STEPS) * BLK)
    s_ref[...] += se_a + se_b
    lsc_ref[...] += lsum_a + lsum_b

    @pl.when(i == NUM_STEPS - 1)
    def _finish():
        lse = jnp.log(s_ref[...])
        nll = lse - lsc_ref[...]
        valid = (labels != IGNORE_INDEX).astype(jnp.float32)
        loss = (jnp.sum(nll * valid, keepdims=True)
                / jnp.maximum(jnp.sum(valid, keepdims=True), 1.0))
        out_ref[...] = loss.reshape(1, 1)


def kernel(inputs, roi_label, roi_ious, lut, cq, reliability):
    del roi_ious
    labels = (roi_label.reshape(-1) - 1).astype(jnp.int32).reshape(BATCH, 1)
    rel_lut = reliability[:NUM_PIDS].reshape(2 * NUM_STEPS, 1, BLK)
    rel_cq = reliability[NUM_PIDS:].reshape(1, NUM_CQ)

    out = pl.pallas_call(
        _oim_kernel,
        grid=(NUM_STEPS,),
        in_specs=[
            pl.BlockSpec((BATCH, NUM_FEATURES), lambda i: (0, 0)),
            pl.BlockSpec((BATCH, 1), lambda i: (0, 0)),
            pl.BlockSpec((BLK, NUM_FEATURES), lambda i: (i, 0)),
            pl.BlockSpec((BLK, NUM_FEATURES),
                         lambda i: (i + NUM_STEPS, 0)),
            pl.BlockSpec((NUM_CQ, NUM_FEATURES), lambda i: (0, 0)),
            pl.BlockSpec((1, 1, BLK), lambda i: (i, 0, 0)),
            pl.BlockSpec((1, 1, BLK),
                         lambda i: (i + NUM_STEPS, 0, 0)),
            pl.BlockSpec((1, NUM_CQ), lambda i: (0, 0)),
        ],
        out_specs=pl.BlockSpec((1, 1), lambda i: (0, 0)),
        out_shape=jax.ShapeDtypeStruct((1, 1), jnp.float32),
        scratch_shapes=[
            pltpu.VMEM((BATCH, 1), jnp.float32),
            pltpu.VMEM((BATCH, 1), jnp.float32),
        ],
    )(inputs, labels, lut, lut, cq, rel_lut, rel_lut, rel_cq)
    return out[0, 0]


# speedup vs baseline: 1.1553x; 1.1553x over previous
import math

import jax
import jax.numpy as jnp
from jax.experimental import pallas as pl
from jax.experimental.pallas import tpu as pltpu

NUM_FEATURES = 128
NUM_PIDS = 100000
NUM_CQ = 5000
OIM_SCALAR = 30.0
BATCH = 128
BLK = 25000
NUM_BLOCKS = NUM_PIDS // BLK
IGNORE_INDEX = 5554
LOG2E = math.log2(math.e)
LN2 = math.log(2.0)


def _oim_kernel(x_ref, lab_ref, lut_ref, cq_ref, relc_lut_ref, relc_cq_ref,
                out_ref, s_ref, lsc_ref):
    i = pl.program_id(0)
    x = x_ref[...]
    labels = lab_ref[...]

    def scores2(w, relc):
        lg = jax.lax.dot_general(
            x, w, (((1,), (1,)), ((), ())),
            preferred_element_type=jnp.float32,
            precision=jax.lax.Precision.DEFAULT)
        return lg * relc

    @pl.when(i == 0)
    def _init():
        ys = scores2(cq_ref[...], relc_cq_ref[...])
        s_ref[...] = jnp.sum(jnp.exp2(ys), axis=1, keepdims=True)
        lsc_ref[...] = jnp.zeros_like(lsc_ref)

    y = scores2(lut_ref[...], relc_lut_ref[0])
    s_ref[...] += jnp.sum(jnp.exp2(y), axis=1, keepdims=True)

    col = jax.lax.broadcasted_iota(jnp.int32, (BATCH, BLK), 1)
    hit = col == labels - i * BLK
    lsc_ref[...] += jnp.sum(jnp.where(hit, y, 0.0), axis=1, keepdims=True)

    @pl.when(i == NUM_BLOCKS - 1)
    def _finish():
        lse = jnp.log(s_ref[...])
        nll = lse - lsc_ref[...] * LN2
        valid = (labels != IGNORE_INDEX).astype(jnp.float32)
        loss = (jnp.sum(nll * valid, keepdims=True)
                / jnp.maximum(jnp.sum(valid, keepdims=True), 1.0))
        out_ref[...] = loss.reshape(1, 1)


def kernel(inputs, roi_label, roi_ious, lut, cq, reliability):
    del roi_ious
    labels = (roi_label.reshape(-1) - 1).astype(jnp.int32).reshape(BATCH, 1)
    relc = reliability * jnp.float32(OIM_SCALAR * LOG2E)
    relc_lut = relc[:NUM_PIDS].reshape(NUM_BLOCKS, 1, BLK)
    relc_cq = relc[NUM_PIDS:].reshape(1, NUM_CQ)

    out = pl.pallas_call(
        _oim_kernel,
        grid=(NUM_BLOCKS,),
        in_specs=[
            pl.BlockSpec((BATCH, NUM_FEATURES), lambda i: (0, 0)),
            pl.BlockSpec((BATCH, 1), lambda i: (0, 0)),
            pl.BlockSpec((BLK, NUM_FEATURES), lambda i: (i, 0)),
            pl.BlockSpec((NUM_CQ, NUM_FEATURES), lambda i: (0, 0)),
            pl.BlockSpec((1, 1, BLK), lambda i: (i, 0, 0)),
            pl.BlockSpec((1, NUM_CQ), lambda i: (0, 0)),
        ],
        out_specs=pl.BlockSpec((1, 1), lambda i: (0, 0)),
        out_shape=jax.ShapeDtypeStruct((1, 1), jnp.float32),
        scratch_shapes=[
            pltpu.VMEM((BATCH, 1), jnp.float32),
            pltpu.VMEM((BATCH, 1), jnp.float32),
        ],
    )(inputs, labels, lut, cq, relc_lut, relc_cq)
    return out[0, 0]
